# Initial kernel scaffold; baseline (speedup 1.0000x reference)
import jax
import jax.numpy as jnp
from jax.experimental import pallas as pl


def kernel(x, edge_index, Ws1, bs1, Ws2, bs2, Wa1, ba1, Wa2, ba2, Wn1, bn1, Wn2, bn2, Wo1, bo1, Wo2, bo2):
    raise NotImplementedError("write your pallas kernel here")


# speedup vs baseline: 8.1011x; 8.1011x over previous
import functools

import jax
import jax.numpy as jnp
from jax import lax
from jax.experimental import pallas as pl
from jax.experimental.pallas import tpu as pltpu
from jax.experimental.pallas import tpu_sc as plsc


def _leaky(x):
    return jnp.where(x > 0, x, 0.1 * x)


def _bdot(a, b):
    return jnp.dot(a.astype(jnp.bfloat16), b.astype(jnp.bfloat16),
                   preferred_element_type=jnp.float32)


def _mlp_self_body(x_ref, w1_ref, b1_ref, w2_ref, b2_ref, o_ref):
    t = _leaky(_bdot(x_ref[...], w1_ref[...]) + b1_ref[...])
    o_ref[...] = _bdot(t, w2_ref[...]) + b2_ref[...]


def _mlp_self(x, Ws1, bs1, Ws2, bs2, block_rows):
    n, f = x.shape
    h = Ws2.shape[1]
    grid = n // block_rows
    return pl.pallas_call(
        _mlp_self_body,
        grid=(grid,),
        in_specs=[
            pl.BlockSpec((block_rows, f), lambda i: (i, 0)),
            pl.BlockSpec(Ws1.shape, lambda i: (0, 0)),
            pl.BlockSpec(bs1.shape, lambda i: (0, 0)),
            pl.BlockSpec(Ws2.shape, lambda i: (0, 0)),
            pl.BlockSpec(bs2.shape, lambda i: (0, 0)),
        ],
        out_specs=pl.BlockSpec((block_rows, h), lambda i: (i, 0)),
        out_shape=jax.ShapeDtypeStruct((n, h), jnp.float32),
    )(x, Ws1, bs1, Ws2, bs2)


def _make_sc_segsum(n_nodes, n_chunks, chunk, h, nc, ns):
    nw = nc * ns
    rows_per_tile = n_nodes // ns
    zrows = 40
    assert rows_per_tile % zrows == 0 and rows_per_tile % 8 == 0

    mesh = plsc.VectorSubcoreMesh(core_axis_name="c", subcore_axis_name="s")

    @functools.partial(
        pl.kernel,
        out_type=jax.ShapeDtypeStruct((nc, n_nodes, h), jnp.float32),
        mesh=mesh,
        scratch_types=[
            pltpu.VMEM((n_chunks, chunk), jnp.int32),
            pltpu.VMEM((n_chunks, chunk), jnp.int32),
            pltpu.VMEM((chunk, h), jnp.float32),
            pltpu.VMEM((zrows, h), jnp.float32),
            pltpu.VMEM_SHARED((n_nodes, h), jnp.float32),
            pltpu.SemaphoreType.DMA,
        ],
    )
    def segsum(h0_hbm, src_hbm, dst_hbm, out_hbm, sidx, didx, rows, zbuf, acc, sem):
        c = lax.axis_index("c")
        s = lax.axis_index("s")
        wid = s * nc + c

        pltpu.sync_copy(src_hbm.at[wid], sidx)
        pltpu.sync_copy(dst_hbm.at[wid], didx)

        zero = jnp.zeros((16,), jnp.float32)
        for i in range(zrows):
            for l in range(h // 16):
                zbuf[i, pl.ds(16 * l, 16)] = zero
        base = s * rows_per_tile
        for i in range(rows_per_tile // zrows):
            pltpu.sync_copy(zbuf, acc.at[pl.ds(base + i * zrows, zrows)])
        plsc.subcore_barrier()

        def body(j, _):
            pltpu.async_copy(h0_hbm.at[sidx.at[j]], rows, sem).wait()
            pltpu.sync_copy(rows, acc.at[didx.at[j]], add=True)
            return _

        lax.fori_loop(0, n_chunks, body, None)
        plsc.subcore_barrier()

        pltpu.sync_copy(
            acc.at[pl.ds(base, rows_per_tile)],
            out_hbm.at[c, pl.ds(base, rows_per_tile)],
        )

    return segsum


def _tail_body(
    p_ref, h0_ref,
    wa1_ref, ba1_ref, wa2_ref, ba2_ref,
    wn1_ref, bn1_ref, wn2_ref, bn2_ref,
    wo1_ref, bo1_ref, wo2t_ref, bo2_ref,
    o_ref,
):
    neigh = p_ref[0] + p_ref[1]
    t = _leaky(_bdot(neigh, wa1_ref[...]) + ba1_ref[...])
    neigh = _bdot(t, wa2_ref[...]) + ba2_ref[...]
    t = _leaky(_bdot(neigh, wn1_ref[...]) + bn1_ref[...])
    hh = jnp.maximum(_bdot(t, wn2_ref[...]) + bn2_ref[...] + h0_ref[...], 0.0)
    t = _leaky(_bdot(hh, wo1_ref[...]) + bo1_ref[...])
    t = t.astype(jnp.bfloat16).astype(jnp.float32)
    o_ref[...] = (
        jnp.sum(t * wo2t_ref[...], axis=1, keepdims=True) + bo2_ref[...]
    )


def _tail(partials, h0, Wa1, ba1, Wa2, ba2, Wn1, bn1, Wn2, bn2, Wo1, bo1, Wo2t, bo2,
          block_rows):
    n, h = h0.shape
    grid = n // block_rows
    full = lambda a: pl.BlockSpec(a.shape, lambda i: tuple(0 for _ in a.shape))
    return pl.pallas_call(
        _tail_body,
        grid=(grid,),
        in_specs=[
            pl.BlockSpec((2, block_rows, h), lambda i: (0, i, 0)),
            pl.BlockSpec((block_rows, h), lambda i: (i, 0)),
            full(Wa1), full(ba1), full(Wa2), full(ba2),
            full(Wn1), full(bn1), full(Wn2), full(bn2),
            full(Wo1), full(bo1), full(Wo2t), full(bo2),
        ],
        out_specs=pl.BlockSpec((block_rows, 1), lambda i: (i, 0)),
        out_shape=jax.ShapeDtypeStruct((n, 1), jnp.float32),
    )(partials, h0, Wa1, ba1, Wa2, ba2, Wn1, bn1, Wn2, bn2, Wo1, bo1, Wo2t, bo2)


def kernel(x, edge_index, Ws1, bs1, Ws2, bs2, Wa1, ba1, Wa2, ba2,
           Wn1, bn1, Wn2, bn2, Wo1, bo1, Wo2, bo2):
    n, f = x.shape
    e = edge_index.shape[1]
    h = Ws2.shape[1]
    nc, ns = 2, 16
    nw = nc * ns

    ew = e // nw
    chunk = 125
    n_chunks = ew // chunk
    assert n_chunks * chunk == ew and nw * ew == e

    block_rows = 1000

    h0 = _mlp_self(x, Ws1, bs1.reshape(1, -1), Ws2, bs2.reshape(1, -1),
                   block_rows)

    src = edge_index[0].reshape(nw, n_chunks, chunk)
    dst = edge_index[1].reshape(nw, n_chunks, chunk)
    n_pad = ((n + 40 * ns - 1) // (40 * ns)) * (40 * ns)
    partials = _make_sc_segsum(n_pad, n_chunks, chunk, h, nc, ns)(h0, src, dst)

    out = _tail(
        partials, h0,
        Wa1, ba1.reshape(1, -1), Wa2, ba2.reshape(1, -1),
        Wn1, bn1.reshape(1, -1), Wn2, bn2.reshape(1, -1),
        Wo1, bo1.reshape(1, -1), Wo2.T, bo2.reshape(1, -1),
        block_rows,
    )
    return out
